# baseline (speedup 1.0000x reference)
import functools

import jax
import jax.numpy as jnp
from jax import lax
from jax.experimental import pallas as pl
from jax.experimental.pallas import tpu as pltpu
from jax.experimental.pallas import tpu_sc as plsc

_B = 16384
_NCAND = 21
_D = 64
_F = _B * _NCAND

_INFO = plsc.get_sparse_core_info()
_NC = _INFO.num_cores
_NS = _INFO.num_subcores
_NW = _NC * _NS
_UPW = _B // _NW
_FPW = _F // _NW
_CHUNK = 896
_NCHUNK = _FPW // _CHUNK


def _tower_body(users_hbm, feats_hbm, utab_hbm, itab_hbm,
                uout_hbm, fout_hbm,
                uidx_v, idx_v, rows0, rows1,
                usem, fsem, gsem0, gsem1, osem0, osem1):
    rows = (rows0, rows1)
    gsem = (gsem0, gsem1)
    osem = (osem0, osem1)
    wid = lax.axis_index("s") * _NC + lax.axis_index("c")
    ubase = wid * _UPW
    fbase = wid * _FPW

    ucopy = pltpu.async_copy(users_hbm.at[pl.ds(ubase, _UPW)], uidx_v, usem)
    fcopy = pltpu.async_copy(feats_hbm.at[pl.ds(fbase, _FPW)], idx_v, fsem)

    ucopy.wait()
    pltpu.async_copy(utab_hbm.at[uidx_v], rows0.at[pl.ds(0, _UPW)],
                     gsem0).wait()
    outc = [pltpu.async_copy(rows0.at[pl.ds(0, _UPW)],
                             uout_hbm.at[pl.ds(ubase, _UPW)], osem0), None]

    fcopy.wait()
    gath = [None, None]
    for c in range(_NCHUNK):
        b = (c + 1) & 1
        if outc[b] is not None:
            outc[b].wait()
        gath[b] = pltpu.async_copy(
            itab_hbm.at[idx_v.at[pl.ds(c * _CHUNK, _CHUNK)]], rows[b], gsem[b])
        if c >= 1:
            pb = c & 1
            gath[pb].wait()
            outc[pb] = pltpu.async_copy(
                rows[pb], fout_hbm.at[pl.ds(fbase + (c - 1) * _CHUNK, _CHUNK)],
                osem[pb])
    lb = _NCHUNK & 1
    gath[lb].wait()
    outc[lb] = pltpu.async_copy(
        rows[lb], fout_hbm.at[pl.ds(fbase + (_NCHUNK - 1) * _CHUNK, _CHUNK)],
        osem[lb])
    outc[0].wait()
    outc[1].wait()


@jax.jit
def _tower_sc(users, feats_flat, user_table, item_table):
    mesh = plsc.VectorSubcoreMesh(core_axis_name="c", subcore_axis_name="s")
    return pl.kernel(
        _tower_body,
        out_type=(jax.ShapeDtypeStruct((_B, _D), jnp.float32),
                  jax.ShapeDtypeStruct((_F, _D), jnp.float32)),
        mesh=mesh,
        compiler_params=pltpu.CompilerParams(use_tc_tiling_on_sc=False),
        scratch_types=[
            pltpu.VMEM((_UPW,), jnp.int32),
            pltpu.VMEM((_FPW,), jnp.int32),
            pltpu.VMEM((_CHUNK, _D), jnp.float32),
            pltpu.VMEM((_CHUNK, _D), jnp.float32),
            pltpu.SemaphoreType.DMA,
            pltpu.SemaphoreType.DMA,
            pltpu.SemaphoreType.DMA,
            pltpu.SemaphoreType.DMA,
            pltpu.SemaphoreType.DMA,
            pltpu.SemaphoreType.DMA,
        ],
    )(users, feats_flat, user_table, item_table)


def kernel(users, feats, user_table, item_table):
    user_emb, feat_flat = _tower_sc(users, feats.reshape(_F), user_table,
                                    item_table)
    return (user_emb, feat_flat.reshape(_B, _NCAND, _D))


# speedup vs baseline: 1.2269x; 1.2269x over previous
import functools

import jax
import jax.numpy as jnp
from jax import lax
from jax.experimental import pallas as pl
from jax.experimental.pallas import tpu as pltpu
from jax.experimental.pallas import tpu_sc as plsc

_B = 16384
_NCAND = 21
_D = 64
_VU = 1000000
_VI = 100000

_INFO = plsc.get_sparse_core_info()
_NC = _INFO.num_cores
_NS = _INFO.num_subcores
_NW = _NC * _NS
_UPW = _B // _NW
_DPW = _D // _NW
_HALF = _B // 2
_CHUNK = 4096


def _gather_block(trow, fidx, base, obuf):
    def kk_body(kk, _):
        off = base + kk * 128
        for j in range(8):
            idx = fidx[pl.ds(off + 16 * j, 16)]
            v = plsc.load_gather(trow, [idx])
            obuf[kk, pl.ds(16 * j, 16)] = v
        return 0

    lax.fori_loop(0, 32, kk_body, 0)


def _tower_body(users_hbm, featsT_hbm, utab_hbm, itabT_hbm,
                uout_hbm, fout_hbm,
                sem_a, sem_b, sem_c, sem_d, sem_e, sem_f):
    wid = lax.axis_index("s") * _NC + lax.axis_index("c")
    iota16 = lax.iota(jnp.int32, 16)

    def item_phase(trow, fidx0, fidx1, obuf0, obuf1):
        fidx = (fidx0, fidx1)
        fsem = (sem_a, sem_b)
        obuf = (obuf0, obuf1)
        osem = (sem_c, sem_d)

        def fidx_dma(u, b):
            f = u // 2
            h = u % 2
            return pltpu.make_async_copy(
                featsT_hbm.at[f, pl.ds(_HALF * h, _HALF)], fidx[b], fsem[b])

        def obuf_dma(f, to, ti0, r, b):
            return pltpu.make_async_copy(
                obuf[b], fout_hbm.at[f, to, pl.ds(ti0, 32), pl.ds(128 * r, 128)],
                osem[b])

        for dd in range(_DPW):
            d = jnp.int32(_DPW) * wid + dd
            to = d // 8
            r = d % 8
            pltpu.async_copy(itabT_hbm.at[d], trow, sem_e).wait()
            fidx_dma(0, 0).start()
            fidx_dma(1, 1).start()

            def g_body(g, _, dd=dd, to=to, r=r):
                f = g
                for h in range(2):
                    u = 2 * g + h
                    fidx_dma(u, h).wait()
                    for c in range(2):
                        base = _CHUNK * c
                        ti0 = 64 * h + 32 * c
                        if dd == 0 and h == 0:
                            @pl.when(g > 0)
                            def _():
                                obuf_dma(f, to, ti0, r, c).wait()
                        else:
                            obuf_dma(f, to, ti0, r, c).wait()
                        _gather_block(trow, fidx[h], base, obuf[c])
                        obuf_dma(f, to, ti0, r, c).start()
                    @pl.when(u + 2 < 2 * _NCAND)
                    def _():
                        fidx_dma(u + 2, h).start()
                return 0

            lax.fori_loop(0, _NCAND, g_body, 0)
        obuf_dma(0, 0, 0, 0, 0).wait()
        obuf_dma(0, 0, 0, 0, 1).wait()

    pl.run_scoped(
        item_phase,
        pltpu.VMEM((_VI,), jnp.float32),
        pltpu.VMEM((_HALF,), jnp.int32),
        pltpu.VMEM((_HALF,), jnp.int32),
        pltpu.VMEM((32, 128), jnp.float32),
        pltpu.VMEM((32, 128), jnp.float32),
    )

    def user_phase(uidx, urows, ut0, ut1):
        ut = (ut0, ut1)
        usem = (sem_a, sem_b)
        pltpu.async_copy(users_hbm.at[pl.ds(_UPW * wid, _UPW)], uidx,
                         sem_e).wait()
        pltpu.async_copy(utab_hbm.at[uidx], urows, sem_f).wait()

        def ut_dma(to, b):
            return pltpu.make_async_copy(
                ut[b], uout_hbm.at[to, pl.ds(4 * wid, 4)], usem[b])

        for to in range(8):
            b = to % 2
            if to >= 2:
                ut_dma(to - 2, b).wait()

            def m_body(m, _, to=to, b=b):
                tii = m >> 3
                rr = m & 7
                col = jnp.full((16,), 8 * to + rr, jnp.int32)
                for j in range(8):
                    row = iota16 + (tii * 128 + 16 * j)
                    v = plsc.load_gather(urows, [row, col])
                    ut[b][tii, pl.ds(128 * rr + 16 * j, 16)] = v
                return 0

            lax.fori_loop(0, 32, m_body, 0, unroll=False)
            ut_dma(to, b).start()
        ut_dma(6, 0).wait()
        ut_dma(7, 1).wait()

    pl.run_scoped(
        user_phase,
        pltpu.VMEM((_UPW,), jnp.int32),
        pltpu.VMEM((_UPW, _D), jnp.float32),
        pltpu.VMEM((4, 1024), jnp.float32),
        pltpu.VMEM((4, 1024), jnp.float32),
    )


@jax.jit
def _tower_sc(users, featsT, user_table, itemT):
    mesh = plsc.VectorSubcoreMesh(core_axis_name="c", subcore_axis_name="s")
    return pl.kernel(
        _tower_body,
        out_type=(jax.ShapeDtypeStruct((8, 128, 1024), jnp.float32),
                  jax.ShapeDtypeStruct((_NCAND, 8, 128, 1024), jnp.float32)),
        mesh=mesh,
        compiler_params=pltpu.CompilerParams(use_tc_tiling_on_sc=False,
                                             needs_layout_passes=False),
        scratch_types=[
            pltpu.SemaphoreType.DMA,
            pltpu.SemaphoreType.DMA,
            pltpu.SemaphoreType.DMA,
            pltpu.SemaphoreType.DMA,
            pltpu.SemaphoreType.DMA,
            pltpu.SemaphoreType.DMA,
        ],
    )(users, featsT, user_table, itemT)


def kernel(users, feats, user_table, item_table):
    u_t, x_t = _tower_sc(users, feats.T, user_table, item_table.T)
    user_emb = (u_t.reshape(8, 128, 8, 128).transpose(1, 3, 0, 2)
                .reshape(_B, _D))
    feat_emb = (x_t.reshape(_NCAND, 8, 128, 8, 128).transpose(2, 4, 0, 1, 3)
                .reshape(_B, _NCAND, _D))
    return (user_emb, feat_emb)


# speedup vs baseline: 1.3247x; 1.0797x over previous
import functools

import jax
import jax.numpy as jnp
from jax import lax
from jax.experimental import pallas as pl
from jax.experimental.pallas import tpu as pltpu
from jax.experimental.pallas import tpu_sc as plsc

_B = 16384
_NCAND = 21
_D = 64
_VU = 1000000
_VI = 100000

_INFO = plsc.get_sparse_core_info()
_NC = _INFO.num_cores
_NS = _INFO.num_subcores
_NW = _NC * _NS
_UPW = _B // _NW
_DPW = _D // _NW
_HALF = _B // 2
_CHUNK = 4096


def _gather_block(trow, fidx, base, obuf):
    def kk_body(kk, _):
        off = base + kk * 128
        for j in range(8):
            idx = fidx[pl.ds(off + 16 * j, 16)]
            v = plsc.load_gather(trow, [idx])
            obuf[kk, pl.ds(16 * j, 16)] = v
        return 0

    lax.fori_loop(0, 32, kk_body, 0)


def _item_body(featsT_hbm, itabT_hbm, fout_hbm,
               sem_a, sem_b, sem_c, sem_d, sem_e):
    wid = lax.axis_index("s") * _NC + lax.axis_index("c")

    def item_phase(trow, fidx0, fidx1, obuf0, obuf1):
        fidx = (fidx0, fidx1)
        fsem = (sem_a, sem_b)
        obuf = (obuf0, obuf1)
        osem = (sem_c, sem_d)

        def fidx_dma(u, b):
            f = u // 2
            h = u % 2
            return pltpu.make_async_copy(
                featsT_hbm.at[f, pl.ds(_HALF * h, _HALF)], fidx[b], fsem[b])

        def obuf_dma(f, to, ti0, r, b):
            return pltpu.make_async_copy(
                obuf[b], fout_hbm.at[f, to, pl.ds(ti0, 32), pl.ds(128 * r, 128)],
                osem[b])

        for dd in range(_DPW):
            d = jnp.int32(_DPW) * wid + dd
            to = d // 8
            r = d % 8
            pltpu.async_copy(itabT_hbm.at[d], trow, sem_e).wait()
            fidx_dma(0, 0).start()
            fidx_dma(1, 1).start()

            def g_body(g, _, dd=dd, to=to, r=r):
                f = g
                for h in range(2):
                    u = 2 * g + h
                    fidx_dma(u, h).wait()
                    for c in range(2):
                        base = _CHUNK * c
                        ti0 = 64 * h + 32 * c
                        if dd == 0 and h == 0:
                            @pl.when(g > 0)
                            def _():
                                obuf_dma(f, to, ti0, r, c).wait()
                        else:
                            obuf_dma(f, to, ti0, r, c).wait()
                        _gather_block(trow, fidx[h], base, obuf[c])
                        obuf_dma(f, to, ti0, r, c).start()
                    @pl.when(u + 2 < 2 * _NCAND)
                    def _():
                        fidx_dma(u + 2, h).start()
                return 0

            lax.fori_loop(0, _NCAND, g_body, 0)
        obuf_dma(0, 0, 0, 0, 0).wait()
        obuf_dma(0, 0, 0, 0, 1).wait()

    pl.run_scoped(
        item_phase,
        pltpu.VMEM((_VI,), jnp.float32),
        pltpu.VMEM((_HALF,), jnp.int32),
        pltpu.VMEM((_HALF,), jnp.int32),
        pltpu.VMEM((32, 128), jnp.float32),
        pltpu.VMEM((32, 128), jnp.float32),
    )


def _user_body(users_hbm, utab_hbm, uout_hbm,
               sem_a, sem_b, sem_e, sem_f):
    wid = lax.axis_index("s") * _NC + lax.axis_index("c")
    iota16 = lax.iota(jnp.int32, 16)

    def user_phase(uidx, urows, ut0, ut1):
        ut = (ut0, ut1)
        usem = (sem_a, sem_b)
        pltpu.async_copy(users_hbm.at[pl.ds(_UPW * wid, _UPW)], uidx,
                         sem_e).wait()
        pltpu.async_copy(utab_hbm.at[uidx], urows, sem_f).wait()

        def ut_dma(to, b):
            return pltpu.make_async_copy(
                ut[b], uout_hbm.at[to, pl.ds(4 * wid, 4)], usem[b])

        for to in range(8):
            b = to % 2
            if to >= 2:
                ut_dma(to - 2, b).wait()

            def m_body(m, _, to=to, b=b):
                tii = m >> 3
                rr = m & 7
                col = jnp.full((16,), 8 * to + rr, jnp.int32)
                for j in range(8):
                    row = iota16 + (tii * 128 + 16 * j)
                    v = plsc.load_gather(urows, [row, col])
                    ut[b][tii, pl.ds(128 * rr + 16 * j, 16)] = v
                return 0

            lax.fori_loop(0, 32, m_body, 0, unroll=False)
            ut_dma(to, b).start()
        ut_dma(6, 0).wait()
        ut_dma(7, 1).wait()

    pl.run_scoped(
        user_phase,
        pltpu.VMEM((_UPW,), jnp.int32),
        pltpu.VMEM((_UPW, _D), jnp.float32),
        pltpu.VMEM((4, 1024), jnp.float32),
        pltpu.VMEM((4, 1024), jnp.float32),
    )


@jax.jit
def _tower_sc(users, featsT, user_table, itemT):
    mesh = plsc.VectorSubcoreMesh(core_axis_name="c", subcore_axis_name="s")
    x_t = pl.kernel(
        _item_body,
        out_type=jax.ShapeDtypeStruct((_NCAND, 8, 128, 1024), jnp.float32),
        mesh=mesh,
        compiler_params=pltpu.CompilerParams(use_tc_tiling_on_sc=False,
                                             needs_layout_passes=False),
        scratch_types=[pltpu.SemaphoreType.DMA] * 5,
    )(featsT, itemT)
    u_t = pl.kernel(
        _user_body,
        out_type=jax.ShapeDtypeStruct((8, 128, 1024), jnp.float32),
        mesh=mesh,
        compiler_params=pltpu.CompilerParams(use_tc_tiling_on_sc=False,
                                             needs_layout_passes=False),
        scratch_types=[pltpu.SemaphoreType.DMA] * 4,
    )(users, user_table)
    return u_t, x_t


def kernel(users, feats, user_table, item_table):
    u_t, x_t = _tower_sc(users, feats.T, user_table, item_table.T)
    user_emb = (u_t.reshape(8, 128, 8, 128).transpose(1, 3, 0, 2)
                .reshape(_B, _D))
    feat_emb = (x_t.reshape(_NCAND, 8, 128, 8, 128).transpose(2, 4, 0, 1, 3)
                .reshape(_B, _NCAND, _D))
    return (user_emb, feat_emb)
